# baseline (speedup 1.0000x reference)
import functools
import math

import jax
import jax.numpy as jnp
from jax import lax
from jax.experimental import pallas as pl
from jax.experimental.pallas import tpu as pltpu
from jax.experimental.pallas import tpu_sc as plsc

N = 10000
E = 320000
D = 128
H = 128
RANK = 10

NC = 2
NS = 16
K = 80
CH = E // (NC * NS * K)
RPT_STRIDE = 624
RPT_SZ = 640

BLK = 2000

_HI = lax.Precision.DEFAULT


def _lrelu(v):
    return jnp.where(v >= 0, v, 0.01 * v)


@functools.cache
def _sc_segsum_call():
    mesh = plsc.VectorSubcoreMesh(core_axis_name="c", subcore_axis_name="s")

    @functools.partial(
        pl.kernel,
        out_type=jax.ShapeDtypeStruct((NC, N, H), jnp.float32),
        mesh=mesh,
        scratch_types=[
            pltpu.VMEM_SHARED((N, H), jnp.float32),
            pltpu.VMEM((CH * K,), jnp.int32),
            pltpu.VMEM((CH, K), jnp.int32),
            pltpu.VMEM((K, H), jnp.float32),
            pltpu.VMEM((K, H), jnp.float32),
            pltpu.SemaphoreType.DMA,
            pltpu.SemaphoreType.DMA,
            pltpu.SemaphoreType.DMA,
            pltpu.SemaphoreType.DMA,
        ],
    )
    def _sc_segsum(src_hbm, dst_hbm, table_hbm, zeros_hbm, out_hbm,
                   acc, src_v, dst_v, rows0, rows1, gsem0, gsem1,
                   ssem0, ssem1):
        c = lax.axis_index("c")
        s = lax.axis_index("s")
        sl = pl.ds(s * RPT_STRIDE, RPT_SZ)
        pltpu.sync_copy(src_hbm.at[c, s], src_v)
        pltpu.sync_copy(dst_hbm.at[c, s], dst_v)
        pltpu.sync_copy(zeros_hbm.at[sl], acc.at[sl])
        plsc.subcore_barrier()

        def _src_idx(ch):
            return src_v.at[pl.ds(ch * K, K)]

        def _wait_g(rows, sem, ch):
            pltpu.make_async_copy(table_hbm.at[_src_idx(ch)], rows, sem).wait()

        def _wait_s(rows, sem, ch):
            pltpu.make_async_copy(rows, acc.at[dst_v.at[ch]], sem).wait()

        pltpu.async_copy(table_hbm.at[_src_idx(0)], rows0, gsem0)
        pltpu.async_copy(table_hbm.at[_src_idx(1)], rows1, gsem1)

        @pl.loop(0, CH - 1, step=2)
        def _edges(ch):
            _wait_g(rows0, gsem0, ch)
            pltpu.async_copy(rows0, acc.at[dst_v.at[ch]], ssem0, add=True)
            _wait_g(rows1, gsem1, ch + 1)
            pltpu.async_copy(rows1, acc.at[dst_v.at[ch + 1]], ssem1, add=True)

            @pl.when(ch + 2 < CH)
            def _():
                _wait_s(rows0, ssem0, ch)
                pltpu.async_copy(table_hbm.at[_src_idx(ch + 2)], rows0, gsem0)

            @pl.when(ch + 3 < CH)
            def _():
                _wait_s(rows1, ssem1, ch + 1)
                pltpu.async_copy(table_hbm.at[_src_idx(ch + 3)], rows1, gsem1)

        _wait_g(rows0, gsem0, CH - 1)
        pltpu.async_copy(rows0, acc.at[dst_v.at[CH - 1]], ssem0, add=True)
        _wait_s(rows0, ssem0, CH - 1)
        _wait_s(rows1, ssem1, CH - 2)

        plsc.subcore_barrier()
        pltpu.sync_copy(acc.at[sl], out_hbm.at[c, sl])

    return _sc_segsum


def _mod_weight(wvec, aWl, abl, aWr, abr, base_W):
    lT = (jnp.dot(wvec, aWl, precision=_HI) + abl).reshape(RANK, H)
    r = (jnp.dot(wvec, aWr, precision=_HI) + abr).reshape(RANK, H)
    mod = lax.dot_general(lT, r, (((0,), (0,)), ((), ())), precision=_HI)
    Wm = base_W * (mod * (1.0 / math.sqrt(RANK)) + 1.0)
    return Wm / (jnp.sqrt(jnp.sum(Wm * Wm, axis=1, keepdims=True)) + 1e-8)


def _tc_body(p_ref, x_ref, w_ref,
             le_aWl_ref, le_abl_ref, le_aWr_ref, le_abr_ref,
             le_W_ref, le_b_ref,
             cat1_W_ref, cat1_b_ref, node_W_ref, node_b_ref,
             cat2_W_ref, cat2_b_ref,
             f0_aWl_ref, f0_abl_ref, f0_aWr_ref, f0_abr_ref,
             f0_W_ref, f0_b_ref,
             f1_aWl_ref, f1_abl_ref, f1_aWr_ref, f1_abr_ref,
             f1_W_ref, f1_b_ref,
             eb_ref, o_ref):
    wvec = w_ref[...]
    out0 = p_ref[0] + p_ref[1] + eb_ref[...]

    Wle = _mod_weight(wvec, le_aWl_ref[...], le_abl_ref[...],
                      le_aWr_ref[...], le_abr_ref[...], le_W_ref[...])
    h = _lrelu(lax.dot_general(out0, Wle, (((1,), (1,)), ((), ())),
                               precision=_HI) + le_b_ref[...])
    h = h + jnp.dot(h, cat1_W_ref[...], precision=_HI) + cat1_b_ref[...]
    xh = jnp.dot(x_ref[...], node_W_ref[...], precision=_HI) + node_b_ref[...]
    h = h + xh
    h = h + jnp.dot(xh, cat2_W_ref[...], precision=_HI) + cat2_b_ref[...]
    h = _lrelu(h)

    Wf0 = _mod_weight(wvec, f0_aWl_ref[...], f0_abl_ref[...],
                      f0_aWr_ref[...], f0_abr_ref[...], f0_W_ref[...])
    h = _lrelu(lax.dot_general(h, Wf0, (((1,), (1,)), ((), ())),
                               precision=_HI) + f0_b_ref[...])

    Wf1 = _mod_weight(wvec, f1_aWl_ref[...], f1_abl_ref[...],
                      f1_aWr_ref[...], f1_abr_ref[...], f1_W_ref[...])
    h = _lrelu(lax.dot_general(h, Wf1, (((1,), (1,)), ((), ())),
                               precision=_HI) + f1_b_ref[...])
    o_ref[...] = h


def _full(shape):
    return pl.BlockSpec(shape, lambda i: tuple(0 for _ in shape))


_TC_IN_SPECS = [
    pl.BlockSpec((NC, BLK, H), lambda i: (0, i, 0)),
    pl.BlockSpec((BLK, D), lambda i: (i, 0)),
    _full((1, D)),
    _full((D, RANK * H)), _full((1, RANK * H)),
    _full((D, RANK * H)), _full((1, RANK * H)),
    _full((H, H)), _full((1, H)),
    _full((H, H)), _full((1, H)),
    _full((D, H)), _full((1, H)),
    _full((H, H)), _full((1, H)),
    _full((D, RANK * H)), _full((1, RANK * H)),
    _full((D, RANK * H)), _full((1, RANK * H)),
    _full((H, H)), _full((1, H)),
    _full((D, RANK * H)), _full((1, RANK * H)),
    _full((D, RANK * H)), _full((1, RANK * H)),
    _full((H, H)), _full((1, H)),
    _full((1, H)),
]

_TC_OUT_SPEC = pl.BlockSpec((BLK, H), lambda i: (i, 0))


def _style_parts(aW, ab):
    half = H * RANK
    aWl = aW[:, :half].reshape(D, H, RANK).transpose(0, 2, 1).reshape(D, half)
    abl = ab[:half].reshape(H, RANK).T.reshape(1, half)
    aWr = aW[:, half:]
    abr = ab[half:].reshape(1, half)
    return aWl, abl, aWr, abr


def kernel(x, edge_index, w, edge_w, edge_b, le_aW, le_ab, le_W, le_b, le_ns,
           node_W, node_b, cat1_W, cat1_b, cat2_W, cat2_b, f0_aW, f0_ab,
           f0_W, f0_b, f0_ns, f1_aW, f1_ab, f1_W, f1_b, f1_ns):
    src = edge_index[0].reshape(NC, NS, CH * K)
    dst = edge_index[1].reshape(NC, NS, CH, K)
    zeros = jnp.zeros((N, H), jnp.float32)
    partials = _sc_segsum_call()(src, dst, edge_w, zeros)

    le_p = _style_parts(le_aW, le_ab)
    f0_p = _style_parts(f0_aW, f0_ab)
    f1_p = _style_parts(f1_aW, f1_ab)

    args = (partials, x, w,
            le_p[0], le_p[1], le_p[2], le_p[3], le_W, le_b.reshape(1, H),
            cat1_W, cat1_b.reshape(1, H), node_W, node_b.reshape(1, H),
            cat2_W, cat2_b.reshape(1, H),
            f0_p[0], f0_p[1], f0_p[2], f0_p[3], f0_W, f0_b.reshape(1, H),
            f1_p[0], f1_p[1], f1_p[2], f1_p[3], f1_W, f1_b.reshape(1, H),
            edge_b.reshape(1, H))
    return pl.pallas_call(
        _tc_body,
        grid=(N // BLK,),
        in_specs=_TC_IN_SPECS,
        out_specs=_TC_OUT_SPEC,
        out_shape=jax.ShapeDtypeStruct((N, H), jnp.float32),
    )(*args)


# speedup vs baseline: 1.0167x; 1.0167x over previous
import functools
import math

import jax
import jax.numpy as jnp
from jax import lax
from jax.experimental import pallas as pl
from jax.experimental.pallas import tpu as pltpu
from jax.experimental.pallas import tpu_sc as plsc

N = 10000
E = 320000
D = 128
H = 128
RANK = 10

NC = 2
NS = 16
K = 80
CH = E // (NC * NS * K)
RPT_STRIDE = 624
RPT_SZ = 640

BLK = 2000

_HI = lax.Precision.DEFAULT


def _lrelu(v):
    return jnp.where(v >= 0, v, 0.01 * v)


@functools.cache
def _sc_segsum_call():
    mesh = plsc.VectorSubcoreMesh(core_axis_name="c", subcore_axis_name="s")

    @functools.partial(
        pl.kernel,
        out_type=jax.ShapeDtypeStruct((NC, N, H), jnp.float32),
        mesh=mesh,
        scratch_types=[
            pltpu.VMEM_SHARED((N, H), jnp.float32),
            pltpu.VMEM((CH * K,), jnp.int32),
            pltpu.VMEM((CH, K), jnp.int32),
            pltpu.VMEM((K, H), jnp.float32),
            pltpu.VMEM((K, H), jnp.float32),
            pltpu.SemaphoreType.DMA,
            pltpu.SemaphoreType.DMA,
            pltpu.SemaphoreType.DMA,
            pltpu.SemaphoreType.DMA,
        ],
    )
    def _sc_segsum(src_hbm, dst_hbm, table_hbm, zeros_hbm, out_hbm,
                   acc, src_v, dst_v, rows0, rows1, gsem0, gsem1,
                   ssem0, ssem1):
        c = lax.axis_index("c")
        s = lax.axis_index("s")
        sl = pl.ds(s * RPT_STRIDE, RPT_SZ)

        def _src_idx(ch):
            return src_v.at[pl.ds(ch * K, K)]

        def _wait_g(rows, sem, ch):
            pltpu.make_async_copy(table_hbm.at[_src_idx(ch)], rows, sem).wait()

        def _wait_s(rows, sem, ch):
            pltpu.make_async_copy(rows, acc.at[dst_v.at[ch]], sem).wait()

        pltpu.sync_copy(src_hbm.at[c, s], src_v)
        pltpu.async_copy(table_hbm.at[_src_idx(0)], rows0, gsem0)
        pltpu.async_copy(table_hbm.at[_src_idx(1)], rows1, gsem1)
        pltpu.sync_copy(dst_hbm.at[c, s], dst_v)
        pltpu.sync_copy(zeros_hbm.at[sl], acc.at[sl])
        plsc.subcore_barrier()

        @pl.loop(0, CH - 1, step=2)
        def _edges(ch):
            _wait_g(rows0, gsem0, ch)
            pltpu.async_copy(rows0, acc.at[dst_v.at[ch]], ssem0, add=True)
            _wait_g(rows1, gsem1, ch + 1)
            pltpu.async_copy(rows1, acc.at[dst_v.at[ch + 1]], ssem1, add=True)

            @pl.when(ch + 2 < CH)
            def _():
                _wait_s(rows0, ssem0, ch)
                pltpu.async_copy(table_hbm.at[_src_idx(ch + 2)], rows0, gsem0)

            @pl.when(ch + 3 < CH)
            def _():
                _wait_s(rows1, ssem1, ch + 1)
                pltpu.async_copy(table_hbm.at[_src_idx(ch + 3)], rows1, gsem1)

        _wait_g(rows0, gsem0, CH - 1)
        pltpu.async_copy(rows0, acc.at[dst_v.at[CH - 1]], ssem0, add=True)
        _wait_s(rows0, ssem0, CH - 1)
        _wait_s(rows1, ssem1, CH - 2)

        plsc.subcore_barrier()
        pltpu.sync_copy(acc.at[sl], out_hbm.at[c, sl])

    return _sc_segsum


def _mod_weight(wvec, aWl, abl, aWr, abr, base_W):
    lT = (jnp.dot(wvec, aWl, precision=_HI) + abl).reshape(RANK, H)
    r = (jnp.dot(wvec, aWr, precision=_HI) + abr).reshape(RANK, H)
    mod = lax.dot_general(lT, r, (((0,), (0,)), ((), ())), precision=_HI)
    Wm = base_W * (mod * (1.0 / math.sqrt(RANK)) + 1.0)
    return Wm / (jnp.sqrt(jnp.sum(Wm * Wm, axis=1, keepdims=True)) + 1e-8)


def _tc_body(p_ref, x_ref, w_ref,
             le_aWl_ref, le_abl_ref, le_aWr_ref, le_abr_ref,
             le_W_ref, le_b_ref,
             cat1_W_ref, cat1_b_ref, node_W_ref, node_b_ref,
             cat2_W_ref, cat2_b_ref,
             f0_aWl_ref, f0_abl_ref, f0_aWr_ref, f0_abr_ref,
             f0_W_ref, f0_b_ref,
             f1_aWl_ref, f1_abl_ref, f1_aWr_ref, f1_abr_ref,
             f1_W_ref, f1_b_ref,
             eb_ref, o_ref, wle_ref, wf0_ref, wf1_ref):
    @pl.when(pl.program_id(0) == 0)
    def _():
        wvec = w_ref[...]
        wle_ref[...] = _mod_weight(wvec, le_aWl_ref[...], le_abl_ref[...],
                                   le_aWr_ref[...], le_abr_ref[...],
                                   le_W_ref[...])
        wf0_ref[...] = _mod_weight(wvec, f0_aWl_ref[...], f0_abl_ref[...],
                                   f0_aWr_ref[...], f0_abr_ref[...],
                                   f0_W_ref[...])
        wf1_ref[...] = _mod_weight(wvec, f1_aWl_ref[...], f1_abl_ref[...],
                                   f1_aWr_ref[...], f1_abr_ref[...],
                                   f1_W_ref[...])

    out0 = p_ref[0] + p_ref[1] + eb_ref[...]
    h = _lrelu(lax.dot_general(out0, wle_ref[...], (((1,), (1,)), ((), ())),
                               precision=_HI) + le_b_ref[...])
    h = h + jnp.dot(h, cat1_W_ref[...], precision=_HI) + cat1_b_ref[...]
    xh = jnp.dot(x_ref[...], node_W_ref[...], precision=_HI) + node_b_ref[...]
    h = h + xh
    h = h + jnp.dot(xh, cat2_W_ref[...], precision=_HI) + cat2_b_ref[...]
    h = _lrelu(h)
    h = _lrelu(lax.dot_general(h, wf0_ref[...], (((1,), (1,)), ((), ())),
                               precision=_HI) + f0_b_ref[...])
    h = _lrelu(lax.dot_general(h, wf1_ref[...], (((1,), (1,)), ((), ())),
                               precision=_HI) + f1_b_ref[...])
    o_ref[...] = h


def _full(shape):
    return pl.BlockSpec(shape, lambda i: tuple(0 for _ in shape))


_TC_IN_SPECS = [
    pl.BlockSpec((NC, BLK, H), lambda i: (0, i, 0)),
    pl.BlockSpec((BLK, D), lambda i: (i, 0)),
    _full((1, D)),
    _full((D, RANK * H)), _full((1, RANK * H)),
    _full((D, RANK * H)), _full((1, RANK * H)),
    _full((H, H)), _full((1, H)),
    _full((H, H)), _full((1, H)),
    _full((D, H)), _full((1, H)),
    _full((H, H)), _full((1, H)),
    _full((D, RANK * H)), _full((1, RANK * H)),
    _full((D, RANK * H)), _full((1, RANK * H)),
    _full((H, H)), _full((1, H)),
    _full((D, RANK * H)), _full((1, RANK * H)),
    _full((D, RANK * H)), _full((1, RANK * H)),
    _full((H, H)), _full((1, H)),
    _full((1, H)),
]

_TC_OUT_SPEC = pl.BlockSpec((BLK, H), lambda i: (i, 0))


def _style_parts(aW, ab):
    half = H * RANK
    aWl = aW[:, :half].reshape(D, H, RANK).transpose(0, 2, 1).reshape(D, half)
    abl = ab[:half].reshape(H, RANK).T.reshape(1, half)
    aWr = aW[:, half:]
    abr = ab[half:].reshape(1, half)
    return aWl, abl, aWr, abr


def kernel(x, edge_index, w, edge_w, edge_b, le_aW, le_ab, le_W, le_b, le_ns,
           node_W, node_b, cat1_W, cat1_b, cat2_W, cat2_b, f0_aW, f0_ab,
           f0_W, f0_b, f0_ns, f1_aW, f1_ab, f1_W, f1_b, f1_ns):
    src = edge_index[0].reshape(NC, NS, CH * K)
    dst = edge_index[1].reshape(NC, NS, CH, K)
    zeros = jnp.zeros((N, H), jnp.float32)
    partials = _sc_segsum_call()(src, dst, edge_w, zeros)

    le_p = _style_parts(le_aW, le_ab)
    f0_p = _style_parts(f0_aW, f0_ab)
    f1_p = _style_parts(f1_aW, f1_ab)

    args = (partials, x, w,
            le_p[0], le_p[1], le_p[2], le_p[3], le_W, le_b.reshape(1, H),
            cat1_W, cat1_b.reshape(1, H), node_W, node_b.reshape(1, H),
            cat2_W, cat2_b.reshape(1, H),
            f0_p[0], f0_p[1], f0_p[2], f0_p[3], f0_W, f0_b.reshape(1, H),
            f1_p[0], f1_p[1], f1_p[2], f1_p[3], f1_W, f1_b.reshape(1, H),
            edge_b.reshape(1, H))
    return pl.pallas_call(
        _tc_body,
        grid=(N // BLK,),
        in_specs=_TC_IN_SPECS,
        out_specs=_TC_OUT_SPEC,
        out_shape=jax.ShapeDtypeStruct((N, H), jnp.float32),
        scratch_shapes=[pltpu.VMEM((H, H), jnp.float32)] * 3,
    )(*args)
